# baseline (speedup 1.0000x reference)
import functools
import math

import jax
import jax.numpy as jnp
from jax import lax
from jax.experimental import pallas as pl
from jax.experimental.pallas import tpu as pltpu
from jax.experimental.pallas import tpu_sc as plsc

N = 10000
E = 320000
H = 128
G = 16
L = 5
K = 64
BN_EPS = 1e-5

NC = 2
NS = 16
NW = NC * NS
CH = 128
EPT = E // NW
NCHUNK = 80
EPAD = NCHUNK * CH
ROWS_PER_TILE = 624
TAIL_ROWS = N - NS * ROWS_PER_TILE

BR = 1000
NBLK = N // BR


def _sc_aggregate(h, src3, dst3, zeros_init):
    mesh = plsc.VectorSubcoreMesh(core_axis_name="c", subcore_axis_name="s")

    @functools.partial(
        pl.kernel,
        out_type=jax.ShapeDtypeStruct((NC, N, H), jnp.float32),
        mesh=mesh,
        scratch_types=[
            pltpu.VMEM((NCHUNK, CH), jnp.int32),
            pltpu.VMEM((NCHUNK, CH), jnp.int32),
            pltpu.VMEM((CH, H), jnp.float32),
            pltpu.VMEM_SHARED((N + 8, H), jnp.float32),
            pltpu.SemaphoreType.DMA,
        ],
    )
    def k(h_hbm, src_hbm, dst_hbm, z_hbm, out_hbm, sidx, didx, rows,
          accum, gsem):
        cid = lax.axis_index("c")
        sid = lax.axis_index("s")
        wid = cid * NS + sid
        row0 = sid * ROWS_PER_TILE
        pltpu.sync_copy(src_hbm.at[wid], sidx)
        pltpu.sync_copy(dst_hbm.at[wid], didx)
        pltpu.sync_copy(z_hbm.at[pl.ds(0, ROWS_PER_TILE)],
                        accum.at[pl.ds(row0, ROWS_PER_TILE)])

        @pl.when(sid == NS - 1)
        def _():
            pltpu.sync_copy(z_hbm.at[pl.ds(0, TAIL_ROWS)],
                            accum.at[pl.ds(NS * ROWS_PER_TILE, TAIL_ROWS)])

        plsc.subcore_barrier()

        def body(i, carry):
            pltpu.async_copy(h_hbm.at[sidx.at[i]], rows, gsem).wait()
            pltpu.sync_copy(rows, accum.at[didx.at[i]], add=True)
            return carry

        lax.fori_loop(0, NCHUNK, body, 0)
        plsc.subcore_barrier()
        pltpu.sync_copy(accum.at[pl.ds(row0, ROWS_PER_TILE)],
                        out_hbm.at[cid, pl.ds(row0, ROWS_PER_TILE)])

        @pl.when(sid == NS - 1)
        def _():
            pltpu.sync_copy(accum.at[pl.ds(NS * ROWS_PER_TILE, TAIL_ROWS)],
                            out_hbm.at[cid, pl.ds(NS * ROWS_PER_TILE, TAIL_ROWS)])

    return k(h, src3, dst3, zeros_init)


def _tc_layer(h, parts, w1, b1r, w2, sr, cr, batch8):
    def body(h_ref, p_ref, w1_ref, b1_ref, w2_ref, s_ref, c_ref, bt_ref,
             ho_ref, pool_ref):
        i = pl.program_id(0)
        z = h_ref[...] + p_ref[0] + p_ref[1]
        a = jnp.dot(z, w1_ref[...], preferred_element_type=jnp.float32)
        a = jnp.maximum(a + b1_ref[...], 0.0)
        z2 = jnp.dot(a, w2_ref[...], preferred_element_type=jnp.float32)
        hn = jnp.maximum(z2 * s_ref[...] + c_ref[...], 0.0)
        ho_ref[...] = hn
        ids = bt_ref[0, 0:1, :]
        seg = lax.broadcasted_iota(jnp.int32, (G, 1), 0)
        oh = (ids == seg).astype(jnp.float32)
        acc = jnp.dot(oh, hn, preferred_element_type=jnp.float32)

        @pl.when(i == 0)
        def _():
            pool_ref[...] = acc

        @pl.when(i > 0)
        def _():
            pool_ref[...] += acc

    return pl.pallas_call(
        body,
        grid=(NBLK,),
        in_specs=[
            pl.BlockSpec((BR, H), lambda i: (i, 0)),
            pl.BlockSpec((NC, BR, H), lambda i: (0, i, 0)),
            pl.BlockSpec((H, H), lambda i: (0, 0)),
            pl.BlockSpec((1, H), lambda i: (0, 0)),
            pl.BlockSpec((H, H), lambda i: (0, 0)),
            pl.BlockSpec((1, H), lambda i: (0, 0)),
            pl.BlockSpec((1, H), lambda i: (0, 0)),
            pl.BlockSpec((1, 8, BR), lambda i: (i, 0, 0)),
        ],
        out_specs=[
            pl.BlockSpec((BR, H), lambda i: (i, 0)),
            pl.BlockSpec((G, H), lambda i: (0, 0)),
        ],
        out_shape=[
            jax.ShapeDtypeStruct((N, H), jnp.float32),
            jax.ShapeDtypeStruct((G, H), jnp.float32),
        ],
    )(h, parts, w1, b1r, w2, sr, cr, batch8)


def _tc_head(emb, fc1W, fc1b, fc2W, fc2b):
    def body(e_ref, w1_ref, b1_ref, w2_ref, b2_ref, o_ref):
        v = jnp.dot(e_ref[...], w1_ref[...], preferred_element_type=jnp.float32)
        v = v + b1_ref[...]
        v = jnp.where(v >= 0.0, v, 0.01 * v)
        u = jnp.dot(v, w2_ref[...], preferred_element_type=jnp.float32)
        u = u + b2_ref[...]
        o_ref[...] = 1.0 / (1.0 + jnp.exp(-u))

    return pl.pallas_call(
        body,
        out_shape=jax.ShapeDtypeStruct((G, K), jnp.float32),
    )(emb, fc1W, fc1b.reshape(1, K * 2), fc2W, fc2b.reshape(1, K))


def kernel(x, edge_index, batch, W1, b1, W2, b2, bn_gamma, bn_beta,
           fc1_W, fc1_b, fc2_W, fc2_b):
    pad = jnp.zeros((NW, EPAD - EPT), jnp.int32)
    src3 = jnp.concatenate(
        [edge_index[0].reshape(NW, EPT), pad], axis=1).reshape(NW, NCHUNK, CH)
    dst3 = jnp.concatenate(
        [edge_index[1].reshape(NW, EPT), pad + N], axis=1).reshape(NW, NCHUNK, CH)
    inv_std = 1.0 / math.sqrt(1.0 + BN_EPS)
    zeros_init = jnp.zeros((ROWS_PER_TILE, H), jnp.float32)
    batch8 = jnp.broadcast_to(batch.reshape(NBLK, 1, BR), (NBLK, 8, BR))
    s_all = bn_gamma * inv_std
    c_all = b2 * s_all + bn_beta

    h = x
    pooled = []
    for l in range(L):
        parts = _sc_aggregate(h, src3, dst3, zeros_init)
        h, pool = _tc_layer(h, parts, W1[l], b1[l].reshape(1, H), W2[l],
                            s_all[l].reshape(1, H), c_all[l].reshape(1, H),
                            batch8)
        pooled.append(pool)
    emb = jnp.concatenate(pooled, axis=1)
    return _tc_head(emb, fc1_W, fc1_b, fc2_W, fc2_b)


# speedup vs baseline: 2.3981x; 2.3981x over previous
import functools
import math

import jax
import jax.numpy as jnp
from jax import lax
from jax.experimental import pallas as pl
from jax.experimental.pallas import tpu as pltpu
from jax.experimental.pallas import tpu_sc as plsc

N = 10000
E = 320000
H = 128
G = 16
L = 5
K = 64
BN_EPS = 1e-5

NC = 2
NS = 16
NW = NC * NS
CH = 128
EPT = E // NW
NCHUNK = 80
EPAD = NCHUNK * CH
ROWS_PER_TILE = 624
TAIL_ROWS = N - NS * ROWS_PER_TILE

BR = 1000
NBLK = N // BR


def _sc_aggregate(h, src3, dst3, zeros_init):
    mesh = plsc.VectorSubcoreMesh(core_axis_name="c", subcore_axis_name="s")

    @functools.partial(
        pl.kernel,
        out_type=jax.ShapeDtypeStruct((NC, N, H), jnp.float32),
        mesh=mesh,
        scratch_types=[
            pltpu.VMEM((NCHUNK, CH), jnp.int32),
            pltpu.VMEM((NCHUNK, CH), jnp.int32),
            pltpu.VMEM((CH, H), jnp.float32),
            pltpu.VMEM_SHARED((N + 256, H), jnp.float32),
            pltpu.SemaphoreType.DMA,
        ],
    )
    def k(h_hbm, src_hbm, dst_hbm, z_hbm, out_hbm, sidx, didx, rows,
          accum, gsem):
        cid = lax.axis_index("c")
        sid = lax.axis_index("s")
        wid = cid * NS + sid
        row0 = sid * ROWS_PER_TILE
        pltpu.sync_copy(src_hbm.at[wid], sidx)
        pltpu.sync_copy(dst_hbm.at[wid], didx)
        pltpu.sync_copy(z_hbm.at[pl.ds(0, ROWS_PER_TILE)],
                        accum.at[pl.ds(row0, ROWS_PER_TILE)])

        @pl.when(sid == NS - 1)
        def _():
            pltpu.sync_copy(z_hbm.at[pl.ds(0, TAIL_ROWS)],
                            accum.at[pl.ds(NS * ROWS_PER_TILE, TAIL_ROWS)])

        plsc.subcore_barrier()

        def body(i, carry):
            pltpu.async_copy(h_hbm.at[sidx.at[i]], rows, gsem).wait()
            pltpu.sync_copy(rows, accum.at[didx.at[i]], add=True)
            return carry

        lax.fori_loop(0, NCHUNK, body, 0)
        plsc.subcore_barrier()
        pltpu.sync_copy(accum.at[pl.ds(row0, ROWS_PER_TILE)],
                        out_hbm.at[cid, pl.ds(row0, ROWS_PER_TILE)])

        @pl.when(sid == NS - 1)
        def _():
            pltpu.sync_copy(accum.at[pl.ds(NS * ROWS_PER_TILE, TAIL_ROWS)],
                            out_hbm.at[cid, pl.ds(NS * ROWS_PER_TILE, TAIL_ROWS)])

    return k(h, src3, dst3, zeros_init)


def _tc_layer(h, parts, w1, b1r, w2, sr, cr, batch8):
    def body(h_ref, p_ref, w1_ref, b1_ref, w2_ref, s_ref, c_ref, bt_ref,
             ho_ref, pool_ref):
        i = pl.program_id(0)
        z = h_ref[...] + p_ref[0] + p_ref[1]
        a = jnp.dot(z, w1_ref[...], preferred_element_type=jnp.float32)
        a = jnp.maximum(a + b1_ref[...], 0.0)
        z2 = jnp.dot(a, w2_ref[...], preferred_element_type=jnp.float32)
        hn = jnp.maximum(z2 * s_ref[...] + c_ref[...], 0.0)
        ho_ref[...] = hn
        ids = bt_ref[0, 0:1, :]
        seg = lax.broadcasted_iota(jnp.int32, (G, 1), 0)
        oh = (ids == seg).astype(jnp.float32)
        acc = jnp.dot(oh, hn, preferred_element_type=jnp.float32)

        @pl.when(i == 0)
        def _():
            pool_ref[...] = acc

        @pl.when(i > 0)
        def _():
            pool_ref[...] += acc

    return pl.pallas_call(
        body,
        grid=(NBLK,),
        in_specs=[
            pl.BlockSpec((BR, H), lambda i: (i, 0)),
            pl.BlockSpec((NC, BR, H), lambda i: (0, i, 0)),
            pl.BlockSpec((H, H), lambda i: (0, 0)),
            pl.BlockSpec((1, H), lambda i: (0, 0)),
            pl.BlockSpec((H, H), lambda i: (0, 0)),
            pl.BlockSpec((1, H), lambda i: (0, 0)),
            pl.BlockSpec((1, H), lambda i: (0, 0)),
            pl.BlockSpec((1, 8, BR), lambda i: (i, 0, 0)),
        ],
        out_specs=[
            pl.BlockSpec((BR, H), lambda i: (i, 0)),
            pl.BlockSpec((G, H), lambda i: (0, 0)),
        ],
        out_shape=[
            jax.ShapeDtypeStruct((N, H), jnp.float32),
            jax.ShapeDtypeStruct((G, H), jnp.float32),
        ],
    )(h, parts, w1, b1r, w2, sr, cr, batch8)


def _tc_head(emb, fc1W, fc1b, fc2W, fc2b):
    def body(e_ref, w1_ref, b1_ref, w2_ref, b2_ref, o_ref):
        v = jnp.dot(e_ref[...], w1_ref[...], preferred_element_type=jnp.float32)
        v = v + b1_ref[...]
        v = jnp.where(v >= 0.0, v, 0.01 * v)
        u = jnp.dot(v, w2_ref[...], preferred_element_type=jnp.float32)
        u = u + b2_ref[...]
        o_ref[...] = 1.0 / (1.0 + jnp.exp(-u))

    return pl.pallas_call(
        body,
        out_shape=jax.ShapeDtypeStruct((G, K), jnp.float32),
    )(emb, fc1W, fc1b.reshape(1, K * 2), fc2W, fc2b.reshape(1, K))


def kernel(x, edge_index, batch, W1, b1, W2, b2, bn_gamma, bn_beta,
           fc1_W, fc1_b, fc2_W, fc2_b):
    npad = EPAD - EPT
    spread = jnp.arange(npad, dtype=jnp.int32)
    src_pad = jnp.broadcast_to((spread * 37) % N, (NW, npad))
    dst_pad = jnp.broadcast_to(N + (spread % 256), (NW, npad))
    src3 = jnp.concatenate(
        [edge_index[0].reshape(NW, EPT), src_pad], axis=1).reshape(NW, NCHUNK, CH)
    dst3 = jnp.concatenate(
        [edge_index[1].reshape(NW, EPT), dst_pad], axis=1).reshape(NW, NCHUNK, CH)
    inv_std = 1.0 / math.sqrt(1.0 + BN_EPS)
    zeros_init = jnp.zeros((ROWS_PER_TILE, H), jnp.float32)
    batch8 = jnp.broadcast_to(batch.reshape(NBLK, 1, BR), (NBLK, 8, BR))
    s_all = bn_gamma * inv_std
    c_all = b2 * s_all + bn_beta

    h = x
    pooled = []
    for l in range(L):
        parts = _sc_aggregate(h, src3, dst3, zeros_init)
        h, pool = _tc_layer(h, parts, W1[l], b1[l].reshape(1, H), W2[l],
                            s_all[l].reshape(1, H), c_all[l].reshape(1, H),
                            batch8)
        pooled.append(pool)
    emb = jnp.concatenate(pooled, axis=1)
    return _tc_head(emb, fc1_W, fc1_b, fc2_W, fc2_b)
